# baseline (speedup 1.0000x reference)
import functools

import jax
import jax.numpy as jnp
from jax import lax
from jax.experimental import pallas as pl
from jax.experimental.pallas import tpu as pltpu
from jax.experimental.pallas import tpu_sc as plsc

E = 8
TOP_K = 2
DIM = 768
HID = 512
NTOK = 2048
NP = NTOK * TOP_K
LANES = 128
BLK = 128
S = NP + E * BLK
NBLK_R = S // BLK
NBLK_S = NTOK // BLK
NBLK = NBLK_R + NBLK_S
TOT = S + NTOK

NC, NS = 2, 16
NW = NC * NS
GPT = S // NW
XPT = NTOK // NW
GCH = 32


def _router_body(xf_ref, rwt_ref, bias_ref, pos_ref, xs_ref, bmap_ref):
    f32 = jnp.float32
    xf = xf_ref[...]
    logits = jnp.dot(xf, rwt_ref[...], preferred_element_type=f32)
    lane = lax.broadcasted_iota(jnp.int32, (NTOK, LANES), 1)
    valid = lane < E
    neg = f32(-1e30)
    logits = jnp.where(valid, logits, neg)
    m = jnp.max(logits, axis=1, keepdims=True)
    p = jnp.where(valid, jnp.exp(logits - m), 0.0)
    scores = p / jnp.sum(p, axis=1, keepdims=True)

    biased = jnp.where(valid, scores + bias_ref[...], neg)
    m0 = jnp.max(biased, axis=1, keepdims=True)
    sel0 = jnp.min(jnp.where(biased == m0, lane, LANES), axis=1, keepdims=True)
    oh0 = lane == sel0
    biased2 = jnp.where(oh0, neg, biased)
    m1 = jnp.max(biased2, axis=1, keepdims=True)
    sel1 = jnp.min(jnp.where(biased2 == m1, lane, LANES), axis=1, keepdims=True)
    oh1 = lane == sel1
    s0 = jnp.sum(jnp.where(oh0, scores, 0.0), axis=1, keepdims=True)
    s1 = jnp.sum(jnp.where(oh1, scores, 0.0), axis=1, keepdims=True)
    nrm = s0 + s1 + 1e-20
    xs_ref[:NTOK, :] = xf * (s0 / nrm)
    xs_ref[NTOK:, :] = xf * (s1 / nrm)

    onehot = jnp.concatenate([oh0.astype(f32), oh1.astype(f32)], axis=0)
    r = lax.broadcasted_iota(jnp.int32, (LANES, LANES), 0)
    c = lax.broadcasted_iota(jnp.int32, (LANES, LANES), 1)
    tri = (r >= c).astype(f32)
    run = jnp.zeros((1, LANES), f32)
    rank_parts = []
    for i in range(NP // LANES):
        och = onehot[i * LANES:(i + 1) * LANES]
        cch = jnp.dot(tri, och, preferred_element_type=f32) + run
        rank_parts.append(jnp.sum(cch * och, axis=1, keepdims=True) - 1.0)
        run = run + jnp.sum(och, axis=0, keepdims=True)
    counts = run
    aligned = jnp.ceil(counts / BLK) * BLK
    upper = (r < c).astype(f32)
    base = jnp.dot(aligned, upper, preferred_element_type=f32)
    rank = jnp.concatenate(rank_parts, axis=0)
    base_p = jnp.sum(onehot * base, axis=1, keepdims=True)
    pos_ref[...] = (base_p + rank).astype(jnp.int32)

    base_blk = base / BLK
    bidx = lax.broadcasted_iota(jnp.int32, (NBLK_R, LANES), 0).astype(f32)
    elane = lax.broadcasted_iota(jnp.int32, (NBLK_R, LANES), 1)
    contrib = (bidx >= base_blk) & (elane >= 1) & (elane < E)
    bmap_ref[...] = jnp.sum(contrib.astype(f32), axis=1, keepdims=True).astype(jnp.int32)


_router = pl.pallas_call(
    _router_body,
    out_shape=[
        jax.ShapeDtypeStruct((NP, 1), jnp.int32),
        jax.ShapeDtypeStruct((NP, DIM), jnp.float32),
        jax.ShapeDtypeStruct((NBLK_R, 1), jnp.int32),
    ],
)


PPT = NP // NW


def _dispatch_body(pos_hbm, xs_hbm, rows_hbm, idx_v, buf_v, sem):
    wid = lax.axis_index("s") * NC + lax.axis_index("c")
    pbase = wid * PPT
    pltpu.sync_copy(pos_hbm.at[pl.ds(pbase, PPT)], idx_v)
    pltpu.sync_copy(xs_hbm.at[pl.ds(pbase, PPT)], buf_v)
    pltpu.async_copy(buf_v, rows_hbm.at[idx_v], sem).wait()


@functools.cache
def _dispatch():
    return pl.kernel(
        _dispatch_body,
        out_type=jax.ShapeDtypeStruct((S, DIM), jnp.float32),
        mesh=plsc.VectorSubcoreMesh(core_axis_name="c", subcore_axis_name="s",
                                    num_cores=NC, num_subcores=NS),
        scratch_types=[
            pltpu.VMEM((PPT,), jnp.int32),
            pltpu.VMEM((PPT, DIM), jnp.float32),
            pltpu.SemaphoreType.DMA,
        ],
        compiler_params=pltpu.CompilerParams(needs_layout_passes=False),
    )


def _ffn_body(bmap_ref, rows_ref, xf_ref, w1_ref, w3_ref, w2_ref,
              ws1_ref, ws3_ref, ws2_ref, out_ref):
    f32 = jnp.float32
    i = pl.program_id(0)

    @pl.when(i < NBLK_R)
    def _():
        xin = rows_ref[...]
        a = jnp.dot(xin, w1_ref[0], preferred_element_type=f32)
        b = jnp.dot(xin, w3_ref[0], preferred_element_type=f32)
        h = a * jax.nn.sigmoid(a) * b
        out_ref[...] = jnp.dot(h, w2_ref[0], preferred_element_type=f32)

    @pl.when(i >= NBLK_R)
    def _():
        xin = xf_ref[...]
        dn = (((1,), (1,)), ((), ()))
        a = lax.dot_general(xin, ws1_ref[...], dn, preferred_element_type=f32)
        b = lax.dot_general(xin, ws3_ref[...], dn, preferred_element_type=f32)
        h = a * jax.nn.sigmoid(a) * b
        out_ref[...] = lax.dot_general(h, ws2_ref[...], dn,
                                       preferred_element_type=f32)


_routed_idx = lambda i, bm: (jnp.minimum(i, NBLK_R - 1), 0)
_shared_idx = lambda i, bm: (jnp.maximum(i - NBLK_R, 0), 0)

_ffn = pl.pallas_call(
    _ffn_body,
    grid_spec=pltpu.PrefetchScalarGridSpec(
        num_scalar_prefetch=1,
        grid=(NBLK,),
        in_specs=[
            pl.BlockSpec((BLK, DIM), _routed_idx),
            pl.BlockSpec((BLK, DIM), _shared_idx),
            pl.BlockSpec((1, DIM, HID), lambda i, bm: (bm[i], 0, 0)),
            pl.BlockSpec((1, DIM, HID), lambda i, bm: (bm[i], 0, 0)),
            pl.BlockSpec((1, HID, DIM), lambda i, bm: (bm[i], 0, 0)),
            pl.BlockSpec((HID, DIM), lambda i, bm: (0, 0)),
            pl.BlockSpec((HID, DIM), lambda i, bm: (0, 0)),
            pl.BlockSpec((DIM, HID), lambda i, bm: (0, 0)),
        ],
        out_specs=pl.BlockSpec((BLK, DIM), lambda i, bm: (i, 0)),
    ),
    out_shape=jax.ShapeDtypeStruct((TOT, DIM), jnp.float32),
)


TPT = NTOK // NW
CCH = 32


def _combine_body(big_hbm, pos_hbm, out_hbm,
                  pos0_v, pos1_v, idx_a, idx_b, buf_a, buf_b, buf_c, sem):
    wid = lax.axis_index("s") * NC + lax.axis_index("c")
    tbase = wid * TPT
    pltpu.sync_copy(pos_hbm.at[pl.ds(tbase, TPT)], pos0_v)
    pltpu.sync_copy(pos_hbm.at[pl.ds(NTOK + tbase, TPT)], pos1_v)

    for ci in range(TPT // CCH):
        for k in range(CCH // 16):
            idx_a[pl.ds(k * 16, 16)] = pos0_v[pl.ds(ci * CCH + k * 16, 16)]
            idx_b[pl.ds(k * 16, 16)] = pos1_v[pl.ds(ci * CCH + k * 16, 16)]
        ca = pltpu.async_copy(big_hbm.at[idx_a], buf_a, sem)
        cb = pltpu.async_copy(big_hbm.at[idx_b], buf_b, sem)
        cc = pltpu.async_copy(big_hbm.at[pl.ds(S + tbase + ci * CCH, CCH)],
                              buf_c, sem)
        ca.wait()
        cb.wait()
        cc.wait()

        def row_body(rr, carry):
            def col_body(k, carry2):
                for u in range(3):
                    sl = pl.ds((k * 3 + u) * 16, 16)
                    buf_a[rr, sl] = buf_a[rr, sl] + buf_b[rr, sl] + buf_c[rr, sl]
                return carry2
            lax.fori_loop(0, DIM // 48, col_body, 0)
            return carry
        lax.fori_loop(0, CCH, row_body, 0)
        pltpu.sync_copy(buf_a, out_hbm.at[pl.ds(tbase + ci * CCH, CCH)])


@functools.cache
def _combine():
    return pl.kernel(
        _combine_body,
        out_type=jax.ShapeDtypeStruct((NTOK, DIM), jnp.float32),
        mesh=plsc.VectorSubcoreMesh(core_axis_name="c", subcore_axis_name="s",
                                    num_cores=NC, num_subcores=NS),
        scratch_types=[
            pltpu.VMEM((TPT,), jnp.int32),
            pltpu.VMEM((TPT,), jnp.int32),
            pltpu.VMEM((CCH,), jnp.int32),
            pltpu.VMEM((CCH,), jnp.int32),
            pltpu.VMEM((CCH, DIM), jnp.float32),
            pltpu.VMEM((CCH, DIM), jnp.float32),
            pltpu.VMEM((CCH, DIM), jnp.float32),
            pltpu.SemaphoreType.DMA,
        ],
        compiler_params=pltpu.CompilerParams(needs_layout_passes=False),
    )


@jax.jit
def kernel(x, router_w, w1, w2, w3, ws1, ws2, ws3, expert_bias):
    f32 = jnp.float32
    xf = x.reshape(NTOK, DIM).astype(f32)
    rwt = jnp.zeros((DIM, LANES), f32).at[:, :E].set(router_w.T)
    biasp = jnp.zeros((1, LANES), f32).at[0, :E].set(expert_bias)

    pos2, xs, bmap2 = _router(xf, rwt, biasp)
    pos = pos2.reshape(NP)
    rows = _dispatch()(pos, xs)

    bmap_ext = jnp.concatenate(
        [bmap2.reshape(NBLK_R), jnp.full((NBLK_S,), E - 1, jnp.int32)])
    big = _ffn(bmap_ext, rows, xf, w1, w3, w2, ws1, ws3, ws2)

    out = _combine()(big, pos)
    return out.reshape(1, NTOK, DIM)


# speedup vs baseline: 1.1160x; 1.1160x over previous
import functools

import jax
import jax.numpy as jnp
from jax import lax
from jax.experimental import pallas as pl
from jax.experimental.pallas import tpu as pltpu
from jax.experimental.pallas import tpu_sc as plsc

E = 8
TOP_K = 2
DIM = 768
HID = 512
NTOK = 2048
NP = NTOK * TOP_K
LANES = 128
BLK = 256
S = NP + E * BLK
NBLK_R = S // BLK
NBLK_S = NTOK // BLK
NBLK = NBLK_R + NBLK_S
TOT = S + NTOK

NC, NS = 2, 16
NW = NC * NS
GPT = S // NW
XPT = NTOK // NW
GCH = 32


def _router_body(xf_ref, rwt_ref, bias_ref, pos_ref, xs_ref, bmap_ref):
    f32 = jnp.float32
    xf = xf_ref[...]
    logits = jnp.dot(xf, rwt_ref[...], preferred_element_type=f32)
    lane = lax.broadcasted_iota(jnp.int32, (NTOK, LANES), 1)
    valid = lane < E
    neg = f32(-1e30)
    logits = jnp.where(valid, logits, neg)
    m = jnp.max(logits, axis=1, keepdims=True)
    p = jnp.where(valid, jnp.exp(logits - m), 0.0)
    scores = p / jnp.sum(p, axis=1, keepdims=True)

    biased = jnp.where(valid, scores + bias_ref[...], neg)
    m0 = jnp.max(biased, axis=1, keepdims=True)
    sel0 = jnp.min(jnp.where(biased == m0, lane, LANES), axis=1, keepdims=True)
    oh0 = lane == sel0
    biased2 = jnp.where(oh0, neg, biased)
    m1 = jnp.max(biased2, axis=1, keepdims=True)
    sel1 = jnp.min(jnp.where(biased2 == m1, lane, LANES), axis=1, keepdims=True)
    oh1 = lane == sel1
    s0 = jnp.sum(jnp.where(oh0, scores, 0.0), axis=1, keepdims=True)
    s1 = jnp.sum(jnp.where(oh1, scores, 0.0), axis=1, keepdims=True)
    nrm = s0 + s1 + 1e-20
    xs_ref[:NTOK, :] = xf * (s0 / nrm)
    xs_ref[NTOK:, :] = xf * (s1 / nrm)

    onehot = jnp.concatenate([oh0.astype(f32), oh1.astype(f32)], axis=0)
    r = lax.broadcasted_iota(jnp.int32, (LANES, LANES), 0)
    c = lax.broadcasted_iota(jnp.int32, (LANES, LANES), 1)
    tri = (r >= c).astype(f32)
    run = jnp.zeros((1, LANES), f32)
    rank_parts = []
    for i in range(NP // LANES):
        och = onehot[i * LANES:(i + 1) * LANES]
        cch = jnp.dot(tri, och, preferred_element_type=f32) + run
        rank_parts.append(jnp.sum(cch * och, axis=1, keepdims=True) - 1.0)
        run = run + jnp.sum(och, axis=0, keepdims=True)
    counts = run
    aligned = jnp.ceil(counts / BLK) * BLK
    upper = (r < c).astype(f32)
    base = jnp.dot(aligned, upper, preferred_element_type=f32)
    rank = jnp.concatenate(rank_parts, axis=0)
    base_p = jnp.sum(onehot * base, axis=1, keepdims=True)
    pos_ref[...] = (base_p + rank).astype(jnp.int32)

    base_blk = base / BLK
    bidx = lax.broadcasted_iota(jnp.int32, (NBLK_R, LANES), 0).astype(f32)
    elane = lax.broadcasted_iota(jnp.int32, (NBLK_R, LANES), 1)
    contrib = (bidx >= base_blk) & (elane >= 1) & (elane < E)
    bmap_ref[...] = jnp.sum(contrib.astype(f32), axis=1, keepdims=True).astype(jnp.int32)


_router = pl.pallas_call(
    _router_body,
    out_shape=[
        jax.ShapeDtypeStruct((NP, 1), jnp.int32),
        jax.ShapeDtypeStruct((NP, DIM), jnp.float32),
        jax.ShapeDtypeStruct((NBLK_R, 1), jnp.int32),
    ],
)


PPT = NP // NW


def _dispatch_body(pos_hbm, xs_hbm, rows_hbm, idx_v, buf_v, sem):
    wid = lax.axis_index("s") * NC + lax.axis_index("c")
    pbase = wid * PPT
    pltpu.sync_copy(pos_hbm.at[pl.ds(pbase, PPT)], idx_v)
    pltpu.sync_copy(xs_hbm.at[pl.ds(pbase, PPT)], buf_v)
    pltpu.async_copy(buf_v, rows_hbm.at[idx_v], sem).wait()


@functools.cache
def _dispatch():
    return pl.kernel(
        _dispatch_body,
        out_type=jax.ShapeDtypeStruct((S, DIM), jnp.float32),
        mesh=plsc.VectorSubcoreMesh(core_axis_name="c", subcore_axis_name="s",
                                    num_cores=NC, num_subcores=NS),
        scratch_types=[
            pltpu.VMEM((PPT,), jnp.int32),
            pltpu.VMEM((PPT, DIM), jnp.float32),
            pltpu.SemaphoreType.DMA,
        ],
        compiler_params=pltpu.CompilerParams(needs_layout_passes=False),
    )


def _ffn_body(bmap_ref, rows_ref, xf_ref, w1_ref, w3_ref, w2_ref,
              ws1_ref, ws3_ref, ws2_ref, out_ref):
    f32 = jnp.float32
    i = pl.program_id(0)

    @pl.when(i < NBLK_R)
    def _():
        xin = rows_ref[...]
        a = jnp.dot(xin, w1_ref[0], preferred_element_type=f32)
        b = jnp.dot(xin, w3_ref[0], preferred_element_type=f32)
        h = a * jax.nn.sigmoid(a) * b
        out_ref[...] = jnp.dot(h, w2_ref[0], preferred_element_type=f32)

    @pl.when(i >= NBLK_R)
    def _():
        xin = xf_ref[...]
        dn = (((1,), (1,)), ((), ()))
        a = lax.dot_general(xin, ws1_ref[...], dn, preferred_element_type=f32)
        b = lax.dot_general(xin, ws3_ref[...], dn, preferred_element_type=f32)
        h = a * jax.nn.sigmoid(a) * b
        out_ref[...] = lax.dot_general(h, ws2_ref[...], dn,
                                       preferred_element_type=f32)


_routed_idx = lambda i, bm: (jnp.minimum(i, NBLK_R - 1), 0)
_shared_idx = lambda i, bm: (jnp.maximum(i - NBLK_R, 0), 0)

_ffn = pl.pallas_call(
    _ffn_body,
    grid_spec=pltpu.PrefetchScalarGridSpec(
        num_scalar_prefetch=1,
        grid=(NBLK,),
        in_specs=[
            pl.BlockSpec((BLK, DIM), _routed_idx),
            pl.BlockSpec((BLK, DIM), _shared_idx),
            pl.BlockSpec((1, DIM, HID), lambda i, bm: (bm[i], 0, 0)),
            pl.BlockSpec((1, DIM, HID), lambda i, bm: (bm[i], 0, 0)),
            pl.BlockSpec((1, HID, DIM), lambda i, bm: (bm[i], 0, 0)),
            pl.BlockSpec((HID, DIM), lambda i, bm: (0, 0)),
            pl.BlockSpec((HID, DIM), lambda i, bm: (0, 0)),
            pl.BlockSpec((DIM, HID), lambda i, bm: (0, 0)),
        ],
        out_specs=pl.BlockSpec((BLK, DIM), lambda i, bm: (i, 0)),
    ),
    out_shape=jax.ShapeDtypeStruct((TOT, DIM), jnp.float32),
)


TPT = NTOK // NW
CCH = 32


def _combine_body(big_hbm, pos_hbm, out_hbm,
                  pos0_v, pos1_v, idx_a, idx_b, buf_a, buf_b, buf_c, sem):
    wid = lax.axis_index("s") * NC + lax.axis_index("c")
    tbase = wid * TPT
    pltpu.sync_copy(pos_hbm.at[pl.ds(tbase, TPT)], pos0_v)
    pltpu.sync_copy(pos_hbm.at[pl.ds(NTOK + tbase, TPT)], pos1_v)

    for ci in range(TPT // CCH):
        for k in range(CCH // 16):
            idx_a[pl.ds(k * 16, 16)] = pos0_v[pl.ds(ci * CCH + k * 16, 16)]
            idx_b[pl.ds(k * 16, 16)] = pos1_v[pl.ds(ci * CCH + k * 16, 16)]
        ca = pltpu.async_copy(big_hbm.at[idx_a], buf_a, sem)
        cb = pltpu.async_copy(big_hbm.at[idx_b], buf_b, sem)
        cc = pltpu.async_copy(big_hbm.at[pl.ds(S + tbase + ci * CCH, CCH)],
                              buf_c, sem)
        ca.wait()
        cb.wait()
        cc.wait()

        def row_body(rr, carry):
            def col_body(k, carry2):
                for u in range(3):
                    sl = pl.ds((k * 3 + u) * 16, 16)
                    buf_a[rr, sl] = buf_a[rr, sl] + buf_b[rr, sl] + buf_c[rr, sl]
                return carry2
            lax.fori_loop(0, DIM // 48, col_body, 0)
            return carry
        lax.fori_loop(0, CCH, row_body, 0)
        pltpu.sync_copy(buf_a, out_hbm.at[pl.ds(tbase + ci * CCH, CCH)])


@functools.cache
def _combine():
    return pl.kernel(
        _combine_body,
        out_type=jax.ShapeDtypeStruct((NTOK, DIM), jnp.float32),
        mesh=plsc.VectorSubcoreMesh(core_axis_name="c", subcore_axis_name="s",
                                    num_cores=NC, num_subcores=NS),
        scratch_types=[
            pltpu.VMEM((TPT,), jnp.int32),
            pltpu.VMEM((TPT,), jnp.int32),
            pltpu.VMEM((CCH,), jnp.int32),
            pltpu.VMEM((CCH,), jnp.int32),
            pltpu.VMEM((CCH, DIM), jnp.float32),
            pltpu.VMEM((CCH, DIM), jnp.float32),
            pltpu.VMEM((CCH, DIM), jnp.float32),
            pltpu.SemaphoreType.DMA,
        ],
        compiler_params=pltpu.CompilerParams(needs_layout_passes=False),
    )


@jax.jit
def kernel(x, router_w, w1, w2, w3, ws1, ws2, ws3, expert_bias):
    f32 = jnp.float32
    xf = x.reshape(NTOK, DIM).astype(f32)
    rwt = jnp.zeros((DIM, LANES), f32).at[:, :E].set(router_w.T)
    biasp = jnp.zeros((1, LANES), f32).at[0, :E].set(expert_bias)

    pos2, xs, bmap2 = _router(xf, rwt, biasp)
    pos = pos2.reshape(NP)
    rows = _dispatch()(pos, xs)

    bmap_ext = jnp.concatenate(
        [bmap2.reshape(NBLK_R), jnp.full((NBLK_S,), E - 1, jnp.int32)])
    big = _ffn(bmap_ext, rows, xf, w1, w3, w2, ws1, ws3, ws2)

    out = _combine()(big, pos)
    return out.reshape(1, NTOK, DIM)


# speedup vs baseline: 1.1295x; 1.0122x over previous
import functools

import jax
import jax.numpy as jnp
from jax import lax
from jax.experimental import pallas as pl
from jax.experimental.pallas import tpu as pltpu
from jax.experimental.pallas import tpu_sc as plsc

E = 8
TOP_K = 2
DIM = 768
HID = 512
NTOK = 2048
NP = NTOK * TOP_K
LANES = 128
BLK = 256
S = NP + E * BLK
NBLK_R = S // BLK
NBLK_S = NTOK // BLK
NBLK = NBLK_R + NBLK_S
TOT = S + NTOK

NC, NS = 2, 16
NW = NC * NS
GPT = S // NW
XPT = NTOK // NW
GCH = 32


def _router_body(xf_ref, rwt_ref, bias_ref, pos_ref, xs_ref, bmap_ref):
    f32 = jnp.float32
    xf = xf_ref[...]
    logits = jnp.dot(xf, rwt_ref[...], preferred_element_type=f32)
    lane = lax.broadcasted_iota(jnp.int32, (NTOK, LANES), 1)
    valid = lane < E
    neg = f32(-1e30)
    logits = jnp.where(valid, logits, neg)
    m = jnp.max(logits, axis=1, keepdims=True)
    p = jnp.where(valid, jnp.exp(logits - m), 0.0)
    scores = p / jnp.sum(p, axis=1, keepdims=True)

    biased = jnp.where(valid, scores + bias_ref[...], neg)
    m0 = jnp.max(biased, axis=1, keepdims=True)
    sel0 = jnp.min(jnp.where(biased == m0, lane, LANES), axis=1, keepdims=True)
    oh0 = lane == sel0
    biased2 = jnp.where(oh0, neg, biased)
    m1 = jnp.max(biased2, axis=1, keepdims=True)
    sel1 = jnp.min(jnp.where(biased2 == m1, lane, LANES), axis=1, keepdims=True)
    oh1 = lane == sel1
    s0 = jnp.sum(jnp.where(oh0, scores, 0.0), axis=1, keepdims=True)
    s1 = jnp.sum(jnp.where(oh1, scores, 0.0), axis=1, keepdims=True)
    nrm = s0 + s1 + 1e-20
    xs_ref[:NTOK, :] = xf * (s0 / nrm)
    xs_ref[NTOK:, :] = xf * (s1 / nrm)

    onehot = jnp.concatenate([oh0.astype(f32), oh1.astype(f32)], axis=0)
    r = lax.broadcasted_iota(jnp.int32, (LANES, LANES), 0)
    c = lax.broadcasted_iota(jnp.int32, (LANES, LANES), 1)
    tri = (r >= c).astype(f32)
    run = jnp.zeros((1, LANES), f32)
    rank_parts = []
    for i in range(NP // LANES):
        och = onehot[i * LANES:(i + 1) * LANES]
        cch = jnp.dot(tri, och, preferred_element_type=f32) + run
        rank_parts.append(jnp.sum(cch * och, axis=1, keepdims=True) - 1.0)
        run = run + jnp.sum(och, axis=0, keepdims=True)
    counts = run
    aligned = jnp.ceil(counts / BLK) * BLK
    upper = (r < c).astype(f32)
    base = jnp.dot(aligned, upper, preferred_element_type=f32)
    rank = jnp.concatenate(rank_parts, axis=0)
    base_p = jnp.sum(onehot * base, axis=1, keepdims=True)
    pos_ref[...] = (base_p + rank).astype(jnp.int32)

    base_blk = base / BLK
    bidx = lax.broadcasted_iota(jnp.int32, (NBLK_R + 1, LANES), 0).astype(f32)
    elane = lax.broadcasted_iota(jnp.int32, (NBLK_R + 1, LANES), 1)
    contrib = (bidx >= base_blk) & (elane >= 1) & (elane < E)
    bmap = jnp.sum(contrib.astype(f32), axis=1, keepdims=True)
    nact = jnp.sum(jnp.where(elane == E, base_blk, 0.0), axis=1, keepdims=True)
    brow = lax.broadcasted_iota(jnp.int32, (NBLK_R + 1, 1), 0)
    bmap_ref[...] = jnp.where(brow < NBLK_R, bmap, nact).astype(jnp.int32)


_router = pl.pallas_call(
    _router_body,
    out_shape=[
        jax.ShapeDtypeStruct((NP, 1), jnp.int32),
        jax.ShapeDtypeStruct((NP, DIM), jnp.float32),
        jax.ShapeDtypeStruct((NBLK_R + 1, 1), jnp.int32),
    ],
)


PPT = NP // NW


def _dispatch_body(pos_hbm, xs_hbm, rows_hbm, idx_v, buf_v, sem):
    wid = lax.axis_index("s") * NC + lax.axis_index("c")
    pbase = wid * PPT
    pltpu.sync_copy(pos_hbm.at[pl.ds(pbase, PPT)], idx_v)
    pltpu.sync_copy(xs_hbm.at[pl.ds(pbase, PPT)], buf_v)
    pltpu.async_copy(buf_v, rows_hbm.at[idx_v], sem).wait()


@functools.cache
def _dispatch():
    return pl.kernel(
        _dispatch_body,
        out_type=jax.ShapeDtypeStruct((S, DIM), jnp.float32),
        mesh=plsc.VectorSubcoreMesh(core_axis_name="c", subcore_axis_name="s",
                                    num_cores=NC, num_subcores=NS),
        scratch_types=[
            pltpu.VMEM((PPT,), jnp.int32),
            pltpu.VMEM((PPT, DIM), jnp.float32),
            pltpu.SemaphoreType.DMA,
        ],
        compiler_params=pltpu.CompilerParams(needs_layout_passes=False),
    )


def _ffn_body(bmap_ref, rows_ref, xf_ref, w1_ref, w3_ref, w2_ref,
              ws1_ref, ws3_ref, ws2_ref, out_ref):
    f32 = jnp.float32
    i = pl.program_id(0)

    @pl.when(i < bmap_ref[NBLK])
    def _():
        xin = rows_ref[...]
        a = jnp.dot(xin, w1_ref[0], preferred_element_type=f32)
        b = jnp.dot(xin, w3_ref[0], preferred_element_type=f32)
        h = a * jax.nn.sigmoid(a) * b
        out_ref[...] = jnp.dot(h, w2_ref[0], preferred_element_type=f32)

    @pl.when(i >= NBLK_R)
    def _():
        xin = xf_ref[...]
        dn = (((1,), (1,)), ((), ()))
        a = lax.dot_general(xin, ws1_ref[...], dn, preferred_element_type=f32)
        b = lax.dot_general(xin, ws3_ref[...], dn, preferred_element_type=f32)
        h = a * jax.nn.sigmoid(a) * b
        out_ref[...] = lax.dot_general(h, ws2_ref[...], dn,
                                       preferred_element_type=f32)


_routed_idx = lambda i, bm: (jnp.minimum(i, NBLK_R - 1), 0)
_shared_idx = lambda i, bm: (jnp.maximum(i - NBLK_R, 0), 0)

_ffn = pl.pallas_call(
    _ffn_body,
    grid_spec=pltpu.PrefetchScalarGridSpec(
        num_scalar_prefetch=1,
        grid=(NBLK,),
        in_specs=[
            pl.BlockSpec((BLK, DIM), _routed_idx),
            pl.BlockSpec((BLK, DIM), _shared_idx),
            pl.BlockSpec((1, DIM, HID), lambda i, bm: (bm[i], 0, 0)),
            pl.BlockSpec((1, DIM, HID), lambda i, bm: (bm[i], 0, 0)),
            pl.BlockSpec((1, HID, DIM), lambda i, bm: (bm[i], 0, 0)),
            pl.BlockSpec((HID, DIM), lambda i, bm: (0, 0)),
            pl.BlockSpec((HID, DIM), lambda i, bm: (0, 0)),
            pl.BlockSpec((DIM, HID), lambda i, bm: (0, 0)),
        ],
        out_specs=pl.BlockSpec((BLK, DIM), lambda i, bm: (i, 0)),
    ),
    out_shape=jax.ShapeDtypeStruct((TOT, DIM), jnp.float32),
)


TPT = NTOK // NW
CCH = 32


def _combine_body(big_hbm, pos_hbm, out_hbm,
                  pos0_v, pos1_v, idx_a, idx_b, buf_a, buf_b, buf_c, sem):
    wid = lax.axis_index("s") * NC + lax.axis_index("c")
    tbase = wid * TPT
    pltpu.sync_copy(pos_hbm.at[pl.ds(tbase, TPT)], pos0_v)
    pltpu.sync_copy(pos_hbm.at[pl.ds(NTOK + tbase, TPT)], pos1_v)

    for ci in range(TPT // CCH):
        for k in range(CCH // 16):
            idx_a[pl.ds(k * 16, 16)] = pos0_v[pl.ds(ci * CCH + k * 16, 16)]
            idx_b[pl.ds(k * 16, 16)] = pos1_v[pl.ds(ci * CCH + k * 16, 16)]
        ca = pltpu.async_copy(big_hbm.at[idx_a], buf_a, sem)
        cb = pltpu.async_copy(big_hbm.at[idx_b], buf_b, sem)
        cc = pltpu.async_copy(big_hbm.at[pl.ds(S + tbase + ci * CCH, CCH)],
                              buf_c, sem)
        ca.wait()
        cb.wait()
        cc.wait()

        def row_body(rr, carry):
            def col_body(k, carry2):
                for u in range(3):
                    sl = pl.ds((k * 3 + u) * 16, 16)
                    buf_a[rr, sl] = buf_a[rr, sl] + buf_b[rr, sl] + buf_c[rr, sl]
                return carry2
            lax.fori_loop(0, DIM // 48, col_body, 0)
            return carry
        lax.fori_loop(0, CCH, row_body, 0)
        pltpu.sync_copy(buf_a, out_hbm.at[pl.ds(tbase + ci * CCH, CCH)])


@functools.cache
def _combine():
    return pl.kernel(
        _combine_body,
        out_type=jax.ShapeDtypeStruct((NTOK, DIM), jnp.float32),
        mesh=plsc.VectorSubcoreMesh(core_axis_name="c", subcore_axis_name="s",
                                    num_cores=NC, num_subcores=NS),
        scratch_types=[
            pltpu.VMEM((TPT,), jnp.int32),
            pltpu.VMEM((TPT,), jnp.int32),
            pltpu.VMEM((CCH,), jnp.int32),
            pltpu.VMEM((CCH,), jnp.int32),
            pltpu.VMEM((CCH, DIM), jnp.float32),
            pltpu.VMEM((CCH, DIM), jnp.float32),
            pltpu.VMEM((CCH, DIM), jnp.float32),
            pltpu.SemaphoreType.DMA,
        ],
        compiler_params=pltpu.CompilerParams(needs_layout_passes=False),
    )


@jax.jit
def kernel(x, router_w, w1, w2, w3, ws1, ws2, ws3, expert_bias):
    f32 = jnp.float32
    xf = x.reshape(NTOK, DIM).astype(f32)
    rwt = jnp.zeros((DIM, LANES), f32).at[:, :E].set(router_w.T)
    biasp = jnp.zeros((1, LANES), f32).at[0, :E].set(expert_bias)

    pos2, xs, bmap2 = _router(xf, rwt, biasp)
    pos = pos2.reshape(NP)
    rows = _dispatch()(pos, xs)

    bm = bmap2.reshape(NBLK_R + 1)
    bmap_ext = jnp.concatenate(
        [bm[:NBLK_R], jnp.full((NBLK_S,), E - 1, jnp.int32), bm[NBLK_R:]])
    big = _ffn(bmap_ext, rows, xf, w1, w3, w2, ws1, ws3, ws2)

    out = _combine()(big, pos)
    return out.reshape(1, NTOK, DIM)


# speedup vs baseline: 1.2026x; 1.0647x over previous
import functools

import jax
import jax.numpy as jnp
from jax import lax
from jax.experimental import pallas as pl
from jax.experimental.pallas import tpu as pltpu
from jax.experimental.pallas import tpu_sc as plsc

E = 8
TOP_K = 2
DIM = 768
HID = 512
NTOK = 2048
NP = NTOK * TOP_K
LANES = 128
BLK = 256
S = NP + E * BLK
NBLK_R = S // BLK
NBLK_S = NTOK // BLK
NBLK = NBLK_R + NBLK_S
TOT = S + NTOK

NC, NS = 2, 16
NW = NC * NS
GPT = S // NW
XPT = NTOK // NW
GCH = 32


def _router_body(xf_ref, rwt_ref, bias_ref, pos_ref, xs_ref, bmap_ref):
    f32 = jnp.float32
    xf = xf_ref[...]
    logits = jnp.dot(xf, rwt_ref[...], preferred_element_type=f32)
    lane = lax.broadcasted_iota(jnp.int32, (NTOK, LANES), 1)
    valid = lane < E
    neg = f32(-1e30)
    logits = jnp.where(valid, logits, neg)
    m = jnp.max(logits, axis=1, keepdims=True)
    p = jnp.where(valid, jnp.exp(logits - m), 0.0)
    scores = p / jnp.sum(p, axis=1, keepdims=True)

    biased = jnp.where(valid, scores + bias_ref[...], neg)
    m0 = jnp.max(biased, axis=1, keepdims=True)
    sel0 = jnp.min(jnp.where(biased == m0, lane, LANES), axis=1, keepdims=True)
    oh0 = lane == sel0
    biased2 = jnp.where(oh0, neg, biased)
    m1 = jnp.max(biased2, axis=1, keepdims=True)
    sel1 = jnp.min(jnp.where(biased2 == m1, lane, LANES), axis=1, keepdims=True)
    oh1 = lane == sel1
    s0 = jnp.sum(jnp.where(oh0, scores, 0.0), axis=1, keepdims=True)
    s1 = jnp.sum(jnp.where(oh1, scores, 0.0), axis=1, keepdims=True)
    nrm = s0 + s1 + 1e-20
    xs_ref[:NTOK, :] = xf * (s0 / nrm)
    xs_ref[NTOK:, :] = xf * (s1 / nrm)

    onehot = jnp.concatenate([oh0.astype(f32), oh1.astype(f32)], axis=0)
    r = lax.broadcasted_iota(jnp.int32, (LANES, LANES), 0)
    c = lax.broadcasted_iota(jnp.int32, (LANES, LANES), 1)
    tri = (r >= c).astype(f32)
    run = jnp.zeros((1, LANES), f32)
    rank_parts = []
    for i in range(NP // LANES):
        och = onehot[i * LANES:(i + 1) * LANES]
        cch = jnp.dot(tri, och, preferred_element_type=f32) + run
        rank_parts.append(jnp.sum(cch * och, axis=1, keepdims=True) - 1.0)
        run = run + jnp.sum(och, axis=0, keepdims=True)
    counts = run
    aligned = jnp.ceil(counts / BLK) * BLK
    upper = (r < c).astype(f32)
    base = jnp.dot(aligned, upper, preferred_element_type=f32)
    rank = jnp.concatenate(rank_parts, axis=0)
    base_p = jnp.sum(onehot * base, axis=1, keepdims=True)
    pos_ref[...] = (base_p + rank).astype(jnp.int32)

    base_blk = base / BLK
    bidx = lax.broadcasted_iota(jnp.int32, (NBLK_R + 1, LANES), 0).astype(f32)
    elane = lax.broadcasted_iota(jnp.int32, (NBLK_R + 1, LANES), 1)
    contrib = (bidx >= base_blk) & (elane >= 1) & (elane < E)
    bmap = jnp.sum(contrib.astype(f32), axis=1, keepdims=True)
    nact = jnp.sum(jnp.where(elane == E, base_blk, 0.0), axis=1, keepdims=True)
    brow = lax.broadcasted_iota(jnp.int32, (NBLK_R + 1, 1), 0)
    bmap_ref[...] = jnp.where(brow < NBLK_R, bmap, nact).astype(jnp.int32)


_router = pl.pallas_call(
    _router_body,
    out_shape=[
        jax.ShapeDtypeStruct((NP, 1), jnp.int32),
        jax.ShapeDtypeStruct((NP, DIM), jnp.float32),
        jax.ShapeDtypeStruct((NBLK_R + 1, 1), jnp.int32),
    ],
)


PPT = NP // NW


def _dispatch_body(pos_hbm, xs_hbm, rows_hbm, idx_v, buf_v, sem):
    wid = lax.axis_index("s") * NC + lax.axis_index("c")
    pbase = wid * PPT
    pltpu.sync_copy(pos_hbm.at[pl.ds(pbase, PPT)], idx_v)
    pltpu.sync_copy(xs_hbm.at[pl.ds(pbase, PPT)], buf_v)
    pltpu.async_copy(buf_v, rows_hbm.at[idx_v], sem).wait()


@functools.cache
def _dispatch():
    return pl.kernel(
        _dispatch_body,
        out_type=jax.ShapeDtypeStruct((S, DIM), jnp.float32),
        mesh=plsc.VectorSubcoreMesh(core_axis_name="c", subcore_axis_name="s",
                                    num_cores=NC, num_subcores=NS),
        scratch_types=[
            pltpu.VMEM((PPT,), jnp.int32),
            pltpu.VMEM((PPT, DIM), jnp.float32),
            pltpu.SemaphoreType.DMA,
        ],
        compiler_params=pltpu.CompilerParams(needs_layout_passes=False),
    )


def _ffn_r_body(bmap_ref, rows_ref, w1_ref, w3_ref, w2_ref, out_ref):
    f32 = jnp.float32
    i = pl.program_id(0)

    @pl.when(i < bmap_ref[NBLK_R])
    def _():
        xin = rows_ref[...]
        a = jnp.dot(xin, w1_ref[0], preferred_element_type=f32)
        b = jnp.dot(xin, w3_ref[0], preferred_element_type=f32)
        h = a * jax.nn.sigmoid(a) * b
        out_ref[...] = jnp.dot(h, w2_ref[0], preferred_element_type=f32)


_ffn_r = pl.pallas_call(
    _ffn_r_body,
    grid_spec=pltpu.PrefetchScalarGridSpec(
        num_scalar_prefetch=1,
        grid=(NBLK_R,),
        in_specs=[
            pl.BlockSpec((BLK, DIM), lambda i, bm: (i, 0)),
            pl.BlockSpec((1, DIM, HID), lambda i, bm: (bm[i], 0, 0)),
            pl.BlockSpec((1, DIM, HID), lambda i, bm: (bm[i], 0, 0)),
            pl.BlockSpec((1, HID, DIM), lambda i, bm: (bm[i], 0, 0)),
        ],
        out_specs=pl.BlockSpec((BLK, DIM), lambda i, bm: (i, 0)),
    ),
    out_shape=jax.ShapeDtypeStruct((S, DIM), jnp.float32),
)


def _ffn_s_body(xf_ref, ws1_ref, ws3_ref, ws2_ref, out_ref):
    f32 = jnp.float32
    xin = xf_ref[...]
    dn = (((1,), (1,)), ((), ()))
    a = lax.dot_general(xin, ws1_ref[...], dn, preferred_element_type=f32)
    b = lax.dot_general(xin, ws3_ref[...], dn, preferred_element_type=f32)
    h = a * jax.nn.sigmoid(a) * b
    out_ref[...] = lax.dot_general(h, ws2_ref[...], dn,
                                   preferred_element_type=f32)


_ffn_s = pl.pallas_call(
    _ffn_s_body,
    grid=(NBLK_S,),
    in_specs=[
        pl.BlockSpec((BLK, DIM), lambda i: (i, 0)),
        pl.BlockSpec((HID, DIM), lambda i: (0, 0)),
        pl.BlockSpec((HID, DIM), lambda i: (0, 0)),
        pl.BlockSpec((DIM, HID), lambda i: (0, 0)),
    ],
    out_specs=pl.BlockSpec((BLK, DIM), lambda i: (i, 0)),
    out_shape=jax.ShapeDtypeStruct((NTOK, DIM), jnp.float32),
)


TPT = NTOK // NW
CCH = 16
NCH = TPT // CCH


def _combine_body(big_hbm, sh_hbm, pos_hbm, out_hbm,
                  pos0_v, pos1_v, idx_a, idx_b, bufs_a, bufs_b, bufs_c,
                  sem0, sem1):
    wid = lax.axis_index("s") * NC + lax.axis_index("c")
    tbase = wid * TPT
    pltpu.sync_copy(pos_hbm.at[pl.ds(tbase, TPT)], pos0_v)
    pltpu.sync_copy(pos_hbm.at[pl.ds(NTOK + tbase, TPT)], pos1_v)
    sems = (sem0, sem1)

    def fire(ci):
        d = ci % 2
        sl = pl.ds(d * CCH, CCH)
        idx_a[sl] = pos0_v[pl.ds(ci * CCH, CCH)]
        idx_b[sl] = pos1_v[pl.ds(ci * CCH, CCH)]
        sem = sems[d]
        return (
            pltpu.async_copy(big_hbm.at[idx_a.at[sl]], bufs_a.at[d], sem),
            pltpu.async_copy(big_hbm.at[idx_b.at[sl]], bufs_b.at[d], sem),
            pltpu.async_copy(sh_hbm.at[pl.ds(tbase + ci * CCH, CCH)],
                             bufs_c.at[d], sem),
        )

    cps = fire(0)
    for ci in range(NCH):
        nxt = fire(ci + 1) if ci + 1 < NCH else None
        for c in cps:
            c.wait()
        d = ci % 2
        buf_a, buf_b, buf_c = bufs_a.at[d], bufs_b.at[d], bufs_c.at[d]

        def row_body(rr, carry):
            def col_body(k, carry2):
                for u in range(3):
                    sl = pl.ds((k * 3 + u) * 16, 16)
                    buf_a[rr, sl] = buf_a[rr, sl] + buf_b[rr, sl] + buf_c[rr, sl]
                return carry2
            lax.fori_loop(0, DIM // 48, col_body, 0)
            return carry
        lax.fori_loop(0, CCH, row_body, 0)
        pltpu.sync_copy(buf_a, out_hbm.at[pl.ds(tbase + ci * CCH, CCH)])
        cps = nxt


@functools.cache
def _combine():
    return pl.kernel(
        _combine_body,
        out_type=jax.ShapeDtypeStruct((NTOK, DIM), jnp.float32),
        mesh=plsc.VectorSubcoreMesh(core_axis_name="c", subcore_axis_name="s",
                                    num_cores=NC, num_subcores=NS),
        scratch_types=[
            pltpu.VMEM((TPT,), jnp.int32),
            pltpu.VMEM((TPT,), jnp.int32),
            pltpu.VMEM((2 * CCH,), jnp.int32),
            pltpu.VMEM((2 * CCH,), jnp.int32),
            pltpu.VMEM((2, CCH, DIM), jnp.float32),
            pltpu.VMEM((2, CCH, DIM), jnp.float32),
            pltpu.VMEM((2, CCH, DIM), jnp.float32),
            pltpu.SemaphoreType.DMA,
            pltpu.SemaphoreType.DMA,
        ],
        compiler_params=pltpu.CompilerParams(needs_layout_passes=False),
    )


@jax.jit
def kernel(x, router_w, w1, w2, w3, ws1, ws2, ws3, expert_bias):
    f32 = jnp.float32
    xf = x.reshape(NTOK, DIM).astype(f32)
    rwt = jnp.zeros((DIM, LANES), f32).at[:, :E].set(router_w.T)
    biasp = jnp.zeros((1, LANES), f32).at[0, :E].set(expert_bias)

    pos2, xs, bmap2 = _router(xf, rwt, biasp)
    pos = pos2.reshape(NP)
    rows = _dispatch()(pos, xs)

    shared = _ffn_s(xf, ws1, ws3, ws2)
    big = _ffn_r(bmap2.reshape(NBLK_R + 1), rows, w1, w3, w2)

    out = _combine()(big, shared, pos)
    return out.reshape(1, NTOK, DIM)
